# baseline (speedup 1.0000x reference)
import functools

import jax
import jax.numpy as jnp
from jax import lax
from jax.experimental import pallas as pl
from jax.experimental.pallas import tpu as pltpu
from jax.experimental.pallas import tpu_sc as plsc

N = 10000
E = 320000
D = 128
DE = 4

NC = 2
NS = 16
NW = NC * NS
CHUNK = 40
NCHUNK = 256
EPW = CHUNK * NCHUNK
EPAD = NW * EPW
NPAD = 10240
ROWS_PT = NPAD // NS
ZROWS = 16


def _ef_body(ea_ref, we_ref, out_ref):
    out_ref[...] = jnp.dot(ea_ref[...], we_ref[...],
                           preferred_element_type=jnp.float32)


def _edge_feat(edge_attr, We):
    B = 4096
    return pl.pallas_call(
        _ef_body,
        grid=(EPAD // B,),
        in_specs=[pl.BlockSpec((B, DE), lambda i: (i, 0)),
                  pl.BlockSpec((DE, D), lambda i: (0, 0))],
        out_specs=pl.BlockSpec((B, D), lambda i: (i, 0)),
        out_shape=jax.ShapeDtypeStruct((EPAD, D), jnp.float32),
    )(edge_attr, We)


def _hmm_body(x_ref, w_ref, b_ref, out_ref):
    out_ref[...] = jnp.dot(x_ref[...], w_ref[...],
                           preferred_element_type=jnp.float32) + b_ref[...]


def _hmm(x, W, b2):
    B = 2000
    return pl.pallas_call(
        _hmm_body,
        grid=(N // B,),
        in_specs=[pl.BlockSpec((B, D), lambda i: (i, 0)),
                  pl.BlockSpec((D, D), lambda i: (0, 0)),
                  pl.BlockSpec((1, D), lambda i: (0, 0))],
        out_specs=pl.BlockSpec((B, D), lambda i: (i, 0)),
        out_shape=jax.ShapeDtypeStruct((N, D), jnp.float32),
    )(x, W, b2)


def _upd_common(x_ref, h_ref, a0_ref, a1_ref, g_ref, be_ref, rm_ref, rv_ref):
    u = h_ref[...] + a0_ref[...] + a1_ref[...]
    scale = g_ref[...] * lax.rsqrt(rv_ref[...] + 1e-5)
    u = (u - rm_ref[...]) * scale + be_ref[...]
    u = jnp.maximum(u, 0.0)
    return 0.5 * x_ref[...] + 0.5 * u


def _updmm_body(x_ref, h_ref, a0_ref, a1_ref, g_ref, be_ref, rm_ref, rv_ref,
                w_ref, b_ref, xo_ref, ho_ref):
    xn = _upd_common(x_ref, h_ref, a0_ref, a1_ref, g_ref, be_ref, rm_ref, rv_ref)
    xo_ref[...] = xn
    ho_ref[...] = jnp.dot(xn, w_ref[...],
                          preferred_element_type=jnp.float32) + b_ref[...]


def _upd_last_body(x_ref, h_ref, a0_ref, a1_ref, g_ref, be_ref, rm_ref, rv_ref,
                   xo_ref):
    xo_ref[...] = _upd_common(x_ref, h_ref, a0_ref, a1_ref,
                              g_ref, be_ref, rm_ref, rv_ref)


def _update_mm(x, h, a0, a1, g2, be2, rm2, rv2, W, b2):
    B = 2000
    row = lambda i: (i, 0)
    fixed = lambda i: (0, 0)
    return pl.pallas_call(
        _updmm_body,
        grid=(N // B,),
        in_specs=[pl.BlockSpec((B, D), row), pl.BlockSpec((B, D), row),
                  pl.BlockSpec((B, D), row), pl.BlockSpec((B, D), row),
                  pl.BlockSpec((1, D), fixed), pl.BlockSpec((1, D), fixed),
                  pl.BlockSpec((1, D), fixed), pl.BlockSpec((1, D), fixed),
                  pl.BlockSpec((D, D), fixed), pl.BlockSpec((1, D), fixed)],
        out_specs=[pl.BlockSpec((B, D), row), pl.BlockSpec((B, D), row)],
        out_shape=[jax.ShapeDtypeStruct((N, D), jnp.float32),
                   jax.ShapeDtypeStruct((N, D), jnp.float32)],
    )(x, h, a0, a1, g2, be2, rm2, rv2, W, b2)


def _update_last(x, h, a0, a1, g2, be2, rm2, rv2):
    B = 2000
    row = lambda i: (i, 0)
    fixed = lambda i: (0, 0)
    return pl.pallas_call(
        _upd_last_body,
        grid=(N // B,),
        in_specs=[pl.BlockSpec((B, D), row), pl.BlockSpec((B, D), row),
                  pl.BlockSpec((B, D), row), pl.BlockSpec((B, D), row),
                  pl.BlockSpec((1, D), fixed), pl.BlockSpec((1, D), fixed),
                  pl.BlockSpec((1, D), fixed), pl.BlockSpec((1, D), fixed)],
        out_specs=pl.BlockSpec((B, D), row),
        out_shape=jax.ShapeDtypeStruct((N, D), jnp.float32),
    )(x, h, a0, a1, g2, be2, rm2, rv2)


def _edge_pass_body(h_hbm, src_hbm, dst_hbm, e_hbm, out_hbm,
                    s0, s1, d0, d1, h0, h1, e0, e1, m0, m1, zbuf, agg_sh,
                    sem_is0, sem_is1, sem_id0, sem_id1,
                    sem_h0, sem_h1, sem_e0, sem_e1, sem_s0, sem_s1):
    c = lax.axis_index("c")
    s = lax.axis_index("s")
    wid = s * NC + c
    ebase = wid * EPW
    sb = (s0, s1)
    db = (d0, d1)
    hb = (h0, h1)
    eb = (e0, e1)
    mb = (m0, m1)
    sem_is = (sem_is0, sem_is1)
    sem_id = (sem_id0, sem_id1)
    sem_h = (sem_h0, sem_h1)
    sem_e = (sem_e0, sem_e1)
    sem_s = (sem_s0, sem_s1)

    def zrow(j, _):
        for t in range(D // 16):
            zbuf[j, pl.ds(t * 16, 16)] = jnp.zeros((16,), jnp.float32)
        return 0
    lax.fori_loop(0, ZROWS, zrow, 0)
    for k in range(ROWS_PT // ZROWS):
        pltpu.sync_copy(zbuf, agg_sh.at[pl.ds(s * ROWS_PT + k * ZROWS, ZROWS)])
    plsc.subcore_barrier()

    def issue_isrc(i, p):
        pltpu.async_copy(src_hbm.at[pl.ds(ebase + i * CHUNK, CHUNK)],
                         sb[p], sem_is[p])

    def wait_isrc(p):
        pltpu.make_async_copy(src_hbm.at[pl.ds(0, CHUNK)], sb[p],
                              sem_is[p]).wait()

    def issue_idst(i, p):
        pltpu.async_copy(dst_hbm.at[pl.ds(ebase + i * CHUNK, CHUNK)],
                         db[p], sem_id[p])

    def wait_idst(p):
        pltpu.make_async_copy(dst_hbm.at[pl.ds(0, CHUNK)], db[p],
                              sem_id[p]).wait()

    def issue_ge(i, p):
        pltpu.async_copy(h_hbm.at[sb[p]], hb[p], sem_h[p])
        pltpu.async_copy(e_hbm.at[pl.ds(ebase + i * CHUNK, CHUNK)],
                         eb[p], sem_e[p])

    def wait_ge(p):
        pltpu.make_async_copy(h_hbm.at[sb[p]], hb[p], sem_h[p]).wait()
        pltpu.make_async_copy(e_hbm.at[pl.ds(0, CHUNK)], eb[p], sem_e[p]).wait()

    def compute(p):
        def row(j, _):
            for t in range(D // 16):
                sl = pl.ds(t * 16, 16)
                mb[p][j, sl] = jnp.maximum(hb[p][j, sl] + eb[p][j, sl], 0.0)
            return 0
        lax.fori_loop(0, CHUNK, row, 0)

    def scatter(p):
        pltpu.async_copy(mb[p], agg_sh.at[db[p]], sem_s[p], add=True)

    def wait_scatter(p):
        pltpu.make_async_copy(mb[p], agg_sh.at[db[p]], sem_s[p]).wait()

    issue_isrc(0, 0)
    issue_isrc(1, 1)
    wait_isrc(0)
    issue_ge(0, 0)

    def body(k, _):
        for p in range(2):
            i = 2 * k + p
            pl.when(i >= 2)(lambda: wait_scatter(p))
            issue_idst(i, p)
            wait_ge(p)
            pl.when(i + 2 < NCHUNK)(lambda: issue_isrc(i + 2, p))

            def launch_next():
                wait_isrc((p + 1) % 2)
                issue_ge(i + 1, (p + 1) % 2)
            pl.when(i + 1 < NCHUNK)(launch_next)
            wait_idst(p)
            compute(p)
            scatter(p)
        return 0
    lax.fori_loop(0, NCHUNK // 2, body, 0)
    wait_scatter(0)
    wait_scatter(1)
    plsc.subcore_barrier()

    pltpu.sync_copy(agg_sh.at[pl.ds(s * ROWS_PT, ROWS_PT)],
                    out_hbm.at[c, pl.ds(s * ROWS_PT, ROWS_PT)])


_edge_pass = functools.partial(
    pl.kernel,
    out_type=jax.ShapeDtypeStruct((NC, NPAD, D), jnp.float32),
    mesh=plsc.VectorSubcoreMesh(core_axis_name="c", subcore_axis_name="s"),
    scratch_types=[
        pltpu.VMEM((CHUNK,), jnp.int32),
        pltpu.VMEM((CHUNK,), jnp.int32),
        pltpu.VMEM((CHUNK,), jnp.int32),
        pltpu.VMEM((CHUNK,), jnp.int32),
        pltpu.VMEM((CHUNK, D), jnp.float32),
        pltpu.VMEM((CHUNK, D), jnp.float32),
        pltpu.VMEM((CHUNK, D), jnp.float32),
        pltpu.VMEM((CHUNK, D), jnp.float32),
        pltpu.VMEM((CHUNK, D), jnp.float32),
        pltpu.VMEM((CHUNK, D), jnp.float32),
        pltpu.VMEM((ZROWS, D), jnp.float32),
        pltpu.VMEM_SHARED((NPAD, D), jnp.float32),
        pltpu.SemaphoreType.DMA,
        pltpu.SemaphoreType.DMA,
        pltpu.SemaphoreType.DMA,
        pltpu.SemaphoreType.DMA,
        pltpu.SemaphoreType.DMA,
        pltpu.SemaphoreType.DMA,
        pltpu.SemaphoreType.DMA,
        pltpu.SemaphoreType.DMA,
        pltpu.SemaphoreType.DMA,
        pltpu.SemaphoreType.DMA,
    ],
)(_edge_pass_body)


def kernel(x, edge_index, edge_attr, batch, W, b, We, gamma, beta,
           run_mean, run_var):
    pad = EPAD - E
    src = jnp.concatenate([edge_index[0],
                           jnp.zeros((pad,), edge_index.dtype)])
    dst = jnp.concatenate([edge_index[1],
                           N + (jnp.arange(pad, dtype=edge_index.dtype)
                                % (NPAD - N))])
    edge_attr = jnp.concatenate(
        [edge_attr, jnp.zeros((pad, DE), edge_attr.dtype)])
    b2 = b.reshape(1, D)
    g2 = gamma.reshape(1, D)
    be2 = beta.reshape(1, D)
    rm2 = run_mean.reshape(1, D)
    rv2 = run_var.reshape(1, D)

    e = _edge_feat(edge_attr, We)
    h = _hmm(x, W, b2)
    for i in range(4):
        aggs = _edge_pass(h, src, dst, e)
        a0 = aggs[0, :N]
        a1 = aggs[1, :N]
        if i < 3:
            x, h = _update_mm(x, h, a0, a1, g2, be2, rm2, rv2, W, b2)
        else:
            x = _update_last(x, h, a0, a1, g2, be2, rm2, rv2)
    return x


# speedup vs baseline: 1.0086x; 1.0086x over previous
import functools

import jax
import jax.numpy as jnp
from jax import lax
from jax.experimental import pallas as pl
from jax.experimental.pallas import tpu as pltpu
from jax.experimental.pallas import tpu_sc as plsc

N = 10000
E = 320000
D = 128
DE = 4

NC = 2
NS = 16
NW = NC * NS
CHUNK = 40
NCHUNK = 256
GRP = 16
NGRP = NCHUNK // GRP
EPW = CHUNK * NCHUNK
EPAD = NW * EPW
NPAD = 10240
ROWS_PT = NPAD // NS
ZROWS = 16


def _ef_body(ea_ref, we_ref, out_ref):
    out_ref[...] = jnp.dot(ea_ref[...], we_ref[...],
                           preferred_element_type=jnp.float32)


def _edge_feat(edge_attr, We):
    B = 4096
    return pl.pallas_call(
        _ef_body,
        grid=(EPAD // B,),
        in_specs=[pl.BlockSpec((B, DE), lambda i: (i, 0)),
                  pl.BlockSpec((DE, D), lambda i: (0, 0))],
        out_specs=pl.BlockSpec((B, D), lambda i: (i, 0)),
        out_shape=jax.ShapeDtypeStruct((EPAD, D), jnp.float32),
    )(edge_attr, We)


def _hmm_body(x_ref, w_ref, b_ref, out_ref):
    out_ref[...] = jnp.dot(x_ref[...], w_ref[...],
                           preferred_element_type=jnp.float32) + b_ref[...]


def _hmm(x, W, b2):
    B = 2000
    return pl.pallas_call(
        _hmm_body,
        grid=(N // B,),
        in_specs=[pl.BlockSpec((B, D), lambda i: (i, 0)),
                  pl.BlockSpec((D, D), lambda i: (0, 0)),
                  pl.BlockSpec((1, D), lambda i: (0, 0))],
        out_specs=pl.BlockSpec((B, D), lambda i: (i, 0)),
        out_shape=jax.ShapeDtypeStruct((N, D), jnp.float32),
    )(x, W, b2)


def _upd_common(x_ref, h_ref, a0_ref, a1_ref, g_ref, be_ref, rm_ref, rv_ref):
    u = h_ref[...] + a0_ref[...] + a1_ref[...]
    scale = g_ref[...] * lax.rsqrt(rv_ref[...] + 1e-5)
    u = (u - rm_ref[...]) * scale + be_ref[...]
    u = jnp.maximum(u, 0.0)
    return 0.5 * x_ref[...] + 0.5 * u


def _updmm_body(x_ref, h_ref, a0_ref, a1_ref, g_ref, be_ref, rm_ref, rv_ref,
                w_ref, b_ref, xo_ref, ho_ref):
    xn = _upd_common(x_ref, h_ref, a0_ref, a1_ref, g_ref, be_ref, rm_ref, rv_ref)
    xo_ref[...] = xn
    ho_ref[...] = jnp.dot(xn, w_ref[...],
                          preferred_element_type=jnp.float32) + b_ref[...]


def _upd_last_body(x_ref, h_ref, a0_ref, a1_ref, g_ref, be_ref, rm_ref, rv_ref,
                   xo_ref):
    xo_ref[...] = _upd_common(x_ref, h_ref, a0_ref, a1_ref,
                              g_ref, be_ref, rm_ref, rv_ref)


def _update_mm(x, h, a0, a1, g2, be2, rm2, rv2, W, b2):
    B = 2000
    row = lambda i: (i, 0)
    fixed = lambda i: (0, 0)
    return pl.pallas_call(
        _updmm_body,
        grid=(N // B,),
        in_specs=[pl.BlockSpec((B, D), row), pl.BlockSpec((B, D), row),
                  pl.BlockSpec((B, D), row), pl.BlockSpec((B, D), row),
                  pl.BlockSpec((1, D), fixed), pl.BlockSpec((1, D), fixed),
                  pl.BlockSpec((1, D), fixed), pl.BlockSpec((1, D), fixed),
                  pl.BlockSpec((D, D), fixed), pl.BlockSpec((1, D), fixed)],
        out_specs=[pl.BlockSpec((B, D), row), pl.BlockSpec((B, D), row)],
        out_shape=[jax.ShapeDtypeStruct((N, D), jnp.float32),
                   jax.ShapeDtypeStruct((N, D), jnp.float32)],
    )(x, h, a0, a1, g2, be2, rm2, rv2, W, b2)


def _update_last(x, h, a0, a1, g2, be2, rm2, rv2):
    B = 2000
    row = lambda i: (i, 0)
    fixed = lambda i: (0, 0)
    return pl.pallas_call(
        _upd_last_body,
        grid=(N // B,),
        in_specs=[pl.BlockSpec((B, D), row), pl.BlockSpec((B, D), row),
                  pl.BlockSpec((B, D), row), pl.BlockSpec((B, D), row),
                  pl.BlockSpec((1, D), fixed), pl.BlockSpec((1, D), fixed),
                  pl.BlockSpec((1, D), fixed), pl.BlockSpec((1, D), fixed)],
        out_specs=pl.BlockSpec((B, D), row),
        out_shape=jax.ShapeDtypeStruct((N, D), jnp.float32),
    )(x, h, a0, a1, g2, be2, rm2, rv2)


def _edge_pass_body(h_hbm, src_hbm, dst_hbm, e_hbm, out_hbm,
                    gs0, gs1, gd0, gd1, h0, h1, e0, e1, m0, m1, zbuf, agg_sh,
                    sem_gs0, sem_gs1, sem_gd0, sem_gd1,
                    sem_h0, sem_h1, sem_e0, sem_e1, sem_s0, sem_s1):
    c = lax.axis_index("c")
    s = lax.axis_index("s")
    wid = s * NC + c
    ebase = wid * EPW
    gsb = (gs0, gs1)
    gdb = (gd0, gd1)
    hb = (h0, h1)
    eb = (e0, e1)
    mb = (m0, m1)
    sem_gs = (sem_gs0, sem_gs1)
    sem_gd = (sem_gd0, sem_gd1)
    sem_h = (sem_h0, sem_h1)
    sem_e = (sem_e0, sem_e1)
    sem_s = (sem_s0, sem_s1)

    def zrow(j, _):
        for t in range(D // 16):
            zbuf[j, pl.ds(t * 16, 16)] = jnp.zeros((16,), jnp.float32)
        return 0
    lax.fori_loop(0, ZROWS, zrow, 0)
    for k in range(ROWS_PT // ZROWS):
        pltpu.sync_copy(zbuf, agg_sh.at[pl.ds(s * ROWS_PT + k * ZROWS, ZROWS)])
    plsc.subcore_barrier()

    def issue_gidx(g, q):
        pltpu.async_copy(src_hbm.at[wid * NGRP + g], gsb[q], sem_gs[q])
        pltpu.async_copy(dst_hbm.at[wid * NGRP + g], gdb[q], sem_gd[q])

    def wait_gidx(q):
        pltpu.make_async_copy(src_hbm.at[0], gsb[q], sem_gs[q]).wait()
        pltpu.make_async_copy(dst_hbm.at[0], gdb[q], sem_gd[q]).wait()

    def issue_ge(g, j, qg, p):
        pltpu.async_copy(h_hbm.at[gsb[qg].at[j]], hb[p], sem_h[p])
        pltpu.async_copy(
            e_hbm.at[pl.ds(ebase + (GRP * g + j) * CHUNK, CHUNK)],
            eb[p], sem_e[p])

    def wait_ge(p):
        pltpu.make_async_copy(h_hbm.at[gsb[0].at[0]], hb[p], sem_h[p]).wait()
        pltpu.make_async_copy(e_hbm.at[pl.ds(0, CHUNK)], eb[p], sem_e[p]).wait()

    def compute(p):
        def row(j, _):
            for t in range(D // 16):
                sl = pl.ds(t * 16, 16)
                mb[p][j, sl] = jnp.maximum(hb[p][j, sl] + eb[p][j, sl], 0.0)
            return 0
        lax.fori_loop(0, CHUNK, row, 0)

    def scatter(j, qg, p):
        pltpu.async_copy(mb[p], agg_sh.at[gdb[qg].at[j]], sem_s[p], add=True)

    def wait_scatter(p):
        pltpu.make_async_copy(mb[p], agg_sh.at[gdb[0].at[0]], sem_s[p]).wait()

    def run_chunk(g, j, qg, first_group, last_group):
        p = j % 2
        if not (first_group and j < 2):
            wait_scatter(p)
        if j == 2 and not last_group:
            issue_gidx(g + 1, 1 - qg)
        wait_ge(p)
        if j == GRP - 1 and not last_group:
            wait_gidx(1 - qg)
        if not (last_group and j == GRP - 1):
            nj = (j + 1) % GRP
            ng = g + (1 if j == GRP - 1 else 0)
            nqg = (1 - qg) if j == GRP - 1 else qg
            issue_ge(ng, nj, nqg, 1 - p)
        compute(p)
        scatter(j, qg, p)

    issue_gidx(0, 0)
    wait_gidx(0)
    issue_ge(0, 0, 0, 0)
    for j in range(GRP):
        run_chunk(0, j, 0, True, False)

    def pair(t, _):
        g = 1 + 2 * t
        for j in range(GRP):
            run_chunk(g, j, 1, False, False)
        for j in range(GRP):
            run_chunk(g + 1, j, 0, False, False)
        return 0
    lax.fori_loop(0, (NGRP - 2) // 2, pair, 0)

    for j in range(GRP):
        run_chunk(NGRP - 1, j, 1, False, True)
    wait_scatter(0)
    wait_scatter(1)
    plsc.subcore_barrier()

    pltpu.sync_copy(agg_sh.at[pl.ds(s * ROWS_PT, ROWS_PT)],
                    out_hbm.at[c, pl.ds(s * ROWS_PT, ROWS_PT)])


_edge_pass = functools.partial(
    pl.kernel,
    out_type=jax.ShapeDtypeStruct((NC, NPAD, D), jnp.float32),
    mesh=plsc.VectorSubcoreMesh(core_axis_name="c", subcore_axis_name="s"),
    scratch_types=[
        pltpu.VMEM((GRP, CHUNK), jnp.int32),
        pltpu.VMEM((GRP, CHUNK), jnp.int32),
        pltpu.VMEM((GRP, CHUNK), jnp.int32),
        pltpu.VMEM((GRP, CHUNK), jnp.int32),
        pltpu.VMEM((CHUNK, D), jnp.float32),
        pltpu.VMEM((CHUNK, D), jnp.float32),
        pltpu.VMEM((CHUNK, D), jnp.float32),
        pltpu.VMEM((CHUNK, D), jnp.float32),
        pltpu.VMEM((CHUNK, D), jnp.float32),
        pltpu.VMEM((CHUNK, D), jnp.float32),
        pltpu.VMEM((ZROWS, D), jnp.float32),
        pltpu.VMEM_SHARED((NPAD, D), jnp.float32),
        pltpu.SemaphoreType.DMA,
        pltpu.SemaphoreType.DMA,
        pltpu.SemaphoreType.DMA,
        pltpu.SemaphoreType.DMA,
        pltpu.SemaphoreType.DMA,
        pltpu.SemaphoreType.DMA,
        pltpu.SemaphoreType.DMA,
        pltpu.SemaphoreType.DMA,
        pltpu.SemaphoreType.DMA,
        pltpu.SemaphoreType.DMA,
    ],
)(_edge_pass_body)


def kernel(x, edge_index, edge_attr, batch, W, b, We, gamma, beta,
           run_mean, run_var):
    pad = EPAD - E
    src = jnp.concatenate([edge_index[0],
                           jnp.zeros((pad,), edge_index.dtype)])
    dst = jnp.concatenate([edge_index[1],
                           N + (jnp.arange(pad, dtype=edge_index.dtype)
                                % (NPAD - N))])
    edge_attr = jnp.concatenate(
        [edge_attr, jnp.zeros((pad, DE), edge_attr.dtype)])
    src = src.reshape(NW * NGRP, GRP, CHUNK)
    dst = dst.reshape(NW * NGRP, GRP, CHUNK)
    b2 = b.reshape(1, D)
    g2 = gamma.reshape(1, D)
    be2 = beta.reshape(1, D)
    rm2 = run_mean.reshape(1, D)
    rv2 = run_var.reshape(1, D)

    e = _edge_feat(edge_attr, We)
    h = _hmm(x, W, b2)
    for i in range(4):
        aggs = _edge_pass(h, src, dst, e)
        a0 = aggs[0, :N]
        a1 = aggs[1, :N]
        if i < 3:
            x, h = _update_mm(x, h, a0, a1, g2, be2, rm2, rv2, W, b2)
        else:
            x = _update_last(x, h, a0, a1, g2, be2, rm2, rv2)
    return x


# speedup vs baseline: 1.0901x; 1.0808x over previous
import functools

import jax
import jax.numpy as jnp
from jax import lax
from jax.experimental import pallas as pl
from jax.experimental.pallas import tpu as pltpu
from jax.experimental.pallas import tpu_sc as plsc

N = 10000
E = 320000
D = 128
DE = 4

NC = 2
NS = 16
NW = NC * NS
CHUNK = 40
NCHUNK = 256
GRP = 16
NGRP = NCHUNK // GRP
EPW = CHUNK * NCHUNK
EPAD = NW * EPW
NPAD = 10240
ROWS_PT = NPAD // NS
ZROWS = 16


def _ef_body(ea_ref, we_ref, out_ref):
    out_ref[...] = jnp.dot(ea_ref[...], we_ref[...],
                           preferred_element_type=jnp.float32)


def _edge_feat(edge_attr, We):
    B = 4096
    return pl.pallas_call(
        _ef_body,
        grid=(EPAD // B,),
        in_specs=[pl.BlockSpec((B, DE), lambda i: (i, 0)),
                  pl.BlockSpec((DE, D), lambda i: (0, 0))],
        out_specs=pl.BlockSpec((B, D), lambda i: (i, 0)),
        out_shape=jax.ShapeDtypeStruct((EPAD, D), jnp.float32),
    )(edge_attr, We)


def _hmm_body(x_ref, w_ref, b_ref, out_ref):
    out_ref[...] = jnp.dot(x_ref[...], w_ref[...],
                           preferred_element_type=jnp.float32) + b_ref[...]


def _hmm(x, W, b2):
    B = 2000
    return pl.pallas_call(
        _hmm_body,
        grid=(N // B,),
        in_specs=[pl.BlockSpec((B, D), lambda i: (i, 0)),
                  pl.BlockSpec((D, D), lambda i: (0, 0)),
                  pl.BlockSpec((1, D), lambda i: (0, 0))],
        out_specs=pl.BlockSpec((B, D), lambda i: (i, 0)),
        out_shape=jax.ShapeDtypeStruct((N, D), jnp.float32),
    )(x, W, b2)


def _upd_common(x_ref, h_ref, a0_ref, a1_ref, g_ref, be_ref, rm_ref, rv_ref):
    u = h_ref[...] + a0_ref[...] + a1_ref[...]
    scale = g_ref[...] * lax.rsqrt(rv_ref[...] + 1e-5)
    u = (u - rm_ref[...]) * scale + be_ref[...]
    u = jnp.maximum(u, 0.0)
    return 0.5 * x_ref[...] + 0.5 * u


def _updmm_body(x_ref, h_ref, a0_ref, a1_ref, g_ref, be_ref, rm_ref, rv_ref,
                w_ref, b_ref, xo_ref, ho_ref):
    xn = _upd_common(x_ref, h_ref, a0_ref, a1_ref, g_ref, be_ref, rm_ref, rv_ref)
    xo_ref[...] = xn
    ho_ref[...] = jnp.dot(xn, w_ref[...],
                          preferred_element_type=jnp.float32) + b_ref[...]


def _upd_last_body(x_ref, h_ref, a0_ref, a1_ref, g_ref, be_ref, rm_ref, rv_ref,
                   xo_ref):
    xo_ref[...] = _upd_common(x_ref, h_ref, a0_ref, a1_ref,
                              g_ref, be_ref, rm_ref, rv_ref)


def _update_mm(x, h, a0, a1, g2, be2, rm2, rv2, W, b2):
    B = 2000
    row = lambda i: (i, 0)
    fixed = lambda i: (0, 0)
    return pl.pallas_call(
        _updmm_body,
        grid=(N // B,),
        in_specs=[pl.BlockSpec((B, D), row), pl.BlockSpec((B, D), row),
                  pl.BlockSpec((B, D), row), pl.BlockSpec((B, D), row),
                  pl.BlockSpec((1, D), fixed), pl.BlockSpec((1, D), fixed),
                  pl.BlockSpec((1, D), fixed), pl.BlockSpec((1, D), fixed),
                  pl.BlockSpec((D, D), fixed), pl.BlockSpec((1, D), fixed)],
        out_specs=[pl.BlockSpec((B, D), row), pl.BlockSpec((B, D), row)],
        out_shape=[jax.ShapeDtypeStruct((N, D), jnp.float32),
                   jax.ShapeDtypeStruct((N, D), jnp.float32)],
    )(x, h, a0, a1, g2, be2, rm2, rv2, W, b2)


def _update_last(x, h, a0, a1, g2, be2, rm2, rv2):
    B = 2000
    row = lambda i: (i, 0)
    fixed = lambda i: (0, 0)
    return pl.pallas_call(
        _upd_last_body,
        grid=(N // B,),
        in_specs=[pl.BlockSpec((B, D), row), pl.BlockSpec((B, D), row),
                  pl.BlockSpec((B, D), row), pl.BlockSpec((B, D), row),
                  pl.BlockSpec((1, D), fixed), pl.BlockSpec((1, D), fixed),
                  pl.BlockSpec((1, D), fixed), pl.BlockSpec((1, D), fixed)],
        out_specs=pl.BlockSpec((B, D), row),
        out_shape=jax.ShapeDtypeStruct((N, D), jnp.float32),
    )(x, h, a0, a1, g2, be2, rm2, rv2)


def _edge_pass_body(h_hbm, src_hbm, dst_hbm, e_hbm, out_hbm,
                    gs0, gs1, gd0, gd1, hr0, hr1, hr2, hr3, m0, m1,
                    zbuf, agg_sh,
                    sem_gs0, sem_gs1, sem_gd0, sem_gd1,
                    sem_h0, sem_h1, sem_h2, sem_h3,
                    sem_e0, sem_e1, sem_s0, sem_s1):
    c = lax.axis_index("c")
    s = lax.axis_index("s")
    wid = s * NC + c
    ebase = wid * EPW
    gsb = (gs0, gs1)
    gdb = (gd0, gd1)
    hr = (hr0, hr1, hr2, hr3)
    mb = (m0, m1)
    sem_gs = (sem_gs0, sem_gs1)
    sem_gd = (sem_gd0, sem_gd1)
    sem_h = (sem_h0, sem_h1, sem_h2, sem_h3)
    sem_e = (sem_e0, sem_e1)
    sem_s = (sem_s0, sem_s1)

    def zrow(j, _):
        for t in range(D // 16):
            zbuf[j, pl.ds(t * 16, 16)] = jnp.zeros((16,), jnp.float32)
        return 0
    lax.fori_loop(0, ZROWS, zrow, 0)
    for k in range(ROWS_PT // ZROWS):
        pltpu.sync_copy(zbuf, agg_sh.at[pl.ds(s * ROWS_PT + k * ZROWS, ZROWS)])
    plsc.subcore_barrier()

    def issue_gidx(g, q):
        pltpu.async_copy(src_hbm.at[wid * NGRP + g], gsb[q], sem_gs[q])
        pltpu.async_copy(dst_hbm.at[wid * NGRP + g], gdb[q], sem_gd[q])

    def wait_gidx(q):
        pltpu.make_async_copy(src_hbm.at[0], gsb[q], sem_gs[q]).wait()
        pltpu.make_async_copy(dst_hbm.at[0], gdb[q], sem_gd[q]).wait()

    def issue_h(j, qg, r):
        pltpu.async_copy(h_hbm.at[gsb[qg].at[j]], hr[r], sem_h[r])

    def wait_h(r):
        pltpu.make_async_copy(h_hbm.at[gsb[0].at[0]], hr[r], sem_h[r]).wait()

    def issue_e(g, j, p):
        pltpu.async_copy(
            e_hbm.at[pl.ds(ebase + (GRP * g + j) * CHUNK, CHUNK)],
            mb[p], sem_e[p])

    def wait_e(p):
        pltpu.make_async_copy(e_hbm.at[pl.ds(0, CHUNK)], mb[p],
                              sem_e[p]).wait()

    def compute(p, r):
        def row(j, _):
            for t in range(D // 16):
                sl = pl.ds(t * 16, 16)
                mb[p][j, sl] = jnp.maximum(hr[r][j, sl] + mb[p][j, sl], 0.0)
            return 0
        lax.fori_loop(0, CHUNK, row, 0)

    def scatter(j, qg, p):
        pltpu.async_copy(mb[p], agg_sh.at[gdb[qg].at[j]], sem_s[p], add=True)

    def wait_scatter(p):
        pltpu.make_async_copy(mb[p], agg_sh.at[gdb[0].at[0]], sem_s[p]).wait()

    def run_chunk(g, j, qg, first_group, last_group):
        p = j % 2
        r = j % 4
        if not (first_group and j < 2):
            wait_scatter(p)
        issue_e(g, j, p)
        if j == 2 and not last_group:
            issue_gidx(g + 1, 1 - qg)
        wait_h(r)
        if j == 13 and not last_group:
            wait_gidx(1 - qg)
        if not (last_group and j >= GRP - 3):
            nj = (j + 3) % GRP
            ng = g + (1 if j >= GRP - 3 else 0)
            nqg = (1 - qg) if j >= GRP - 3 else qg
            issue_h_args = (nj, nqg, (j + 3) % 4)
            issue_h(*issue_h_args)
        wait_e(p)
        compute(p, r)
        scatter(j, qg, p)

    issue_gidx(0, 0)
    wait_gidx(0)
    issue_h(0, 0, 0)
    issue_h(1, 0, 1)
    issue_h(2, 0, 2)
    for j in range(GRP):
        run_chunk(0, j, 0, True, False)

    def pair(t, _):
        g = 1 + 2 * t
        for j in range(GRP):
            run_chunk(g, j, 1, False, False)
        for j in range(GRP):
            run_chunk(g + 1, j, 0, False, False)
        return 0
    lax.fori_loop(0, (NGRP - 2) // 2, pair, 0)

    for j in range(GRP):
        run_chunk(NGRP - 1, j, 1, False, True)
    wait_scatter(0)
    wait_scatter(1)
    plsc.subcore_barrier()

    pltpu.sync_copy(agg_sh.at[pl.ds(s * ROWS_PT, ROWS_PT)],
                    out_hbm.at[c, pl.ds(s * ROWS_PT, ROWS_PT)])


_edge_pass = functools.partial(
    pl.kernel,
    out_type=jax.ShapeDtypeStruct((NC, NPAD, D), jnp.float32),
    mesh=plsc.VectorSubcoreMesh(core_axis_name="c", subcore_axis_name="s"),
    scratch_types=[
        pltpu.VMEM((GRP, CHUNK), jnp.int32),
        pltpu.VMEM((GRP, CHUNK), jnp.int32),
        pltpu.VMEM((GRP, CHUNK), jnp.int32),
        pltpu.VMEM((GRP, CHUNK), jnp.int32),
        pltpu.VMEM((CHUNK, D), jnp.float32),
        pltpu.VMEM((CHUNK, D), jnp.float32),
        pltpu.VMEM((CHUNK, D), jnp.float32),
        pltpu.VMEM((CHUNK, D), jnp.float32),
        pltpu.VMEM((CHUNK, D), jnp.float32),
        pltpu.VMEM((CHUNK, D), jnp.float32),
        pltpu.VMEM((ZROWS, D), jnp.float32),
        pltpu.VMEM_SHARED((NPAD, D), jnp.float32),
        pltpu.SemaphoreType.DMA,
        pltpu.SemaphoreType.DMA,
        pltpu.SemaphoreType.DMA,
        pltpu.SemaphoreType.DMA,
        pltpu.SemaphoreType.DMA,
        pltpu.SemaphoreType.DMA,
        pltpu.SemaphoreType.DMA,
        pltpu.SemaphoreType.DMA,
        pltpu.SemaphoreType.DMA,
        pltpu.SemaphoreType.DMA,
        pltpu.SemaphoreType.DMA,
        pltpu.SemaphoreType.DMA,
    ],
)(_edge_pass_body)


def kernel(x, edge_index, edge_attr, batch, W, b, We, gamma, beta,
           run_mean, run_var):
    pad = EPAD - E
    src = jnp.concatenate([edge_index[0],
                           jnp.zeros((pad,), edge_index.dtype)])
    dst = jnp.concatenate([edge_index[1],
                           N + (jnp.arange(pad, dtype=edge_index.dtype)
                                % (NPAD - N))])
    edge_attr = jnp.concatenate(
        [edge_attr, jnp.zeros((pad, DE), edge_attr.dtype)])
    src = src.reshape(NW * NGRP, GRP, CHUNK)
    dst = dst.reshape(NW * NGRP, GRP, CHUNK)
    b2 = b.reshape(1, D)
    g2 = gamma.reshape(1, D)
    be2 = beta.reshape(1, D)
    rm2 = run_mean.reshape(1, D)
    rv2 = run_var.reshape(1, D)

    e = _edge_feat(edge_attr, We)
    h = _hmm(x, W, b2)
    for i in range(4):
        aggs = _edge_pass(h, src, dst, e)
        a0 = aggs[0, :N]
        a1 = aggs[1, :N]
        if i < 3:
            x, h = _update_mm(x, h, a0, a1, g2, be2, rm2, rv2, W, b2)
        else:
            x = _update_last(x, h, a0, a1, g2, be2, rm2, rv2)
    return x
